# Initial kernel scaffold; baseline (speedup 1.0000x reference)
import functools

import jax
import jax.numpy as jnp
from jax import lax
from jax.experimental import pallas as pl
from jax.experimental.pallas import tpu as pltpu
from jax.experimental.pallas import tpu_sc as plsc

NC = 2
NS = 16
LANES = 16
CB = 80


def _largest_divisor_le(n, cap):
    for k in range(min(cap, n), 0, -1):
        if n % k == 0:
            return k
    return 1


def _spmm_sc(feat, row3, col3, ev3):
    n_nodes, d = feat.shape
    nch = row3.shape[1]
    rows_per_sub = n_nodes // NS
    rstg = _largest_divisor_le(rows_per_sub, 128)
    nstg = rows_per_sub // rstg
    mesh = plsc.VectorSubcoreMesh(core_axis_name="c", subcore_axis_name="s")

    @functools.partial(
        pl.kernel,
        out_type=jax.ShapeDtypeStruct((NC, n_nodes, d), jnp.float32),
        mesh=mesh,
        scratch_types=[
            pltpu.VMEM_SHARED((n_nodes, d), jnp.float32),
            pltpu.VMEM((nch, CB), jnp.int32),
            pltpu.VMEM((nch, CB), jnp.int32),
            pltpu.VMEM((nch, CB), jnp.float32),
            pltpu.VMEM((CB, d), jnp.float32),
            pltpu.VMEM((None, d), jnp.float32),
            pltpu.SemaphoreType.DMA,
        ],
    )
    def spmm(feat_hbm, row_hbm, col_hbm, ev_hbm, out_hbm,
             acc, colv, rowv, evv, grows, zstg, sem):
        cid = lax.axis_index("c")
        sid = lax.axis_index("s")
        wid = sid * NC + cid

        zeros16 = jnp.zeros((LANES,), jnp.float32)

        def zb(i, carry):
            for j in range(d // LANES):
                zstg[i, pl.ds(j * LANES, LANES)] = zeros16
            return carry

        lax.fori_loop(0, rstg, zb, 0)
        for k in range(nstg):
            start = sid * rows_per_sub + k * rstg
            pltpu.sync_copy(zstg, acc.at[pl.ds(start, rstg)])

        pltpu.sync_copy(col_hbm.at[wid], colv)
        pltpu.sync_copy(row_hbm.at[wid], rowv)
        pltpu.sync_copy(ev_hbm.at[wid], evv)

        plsc.subcore_barrier()

        def chunk(ch, carry):
            pltpu.async_copy(feat_hbm.at[colv.at[ch]], grows, sem).wait()

            def scale(e, c2):
                w = evv[ch, e]
                for j in range(d // LANES):
                    sl = pl.ds(j * LANES, LANES)
                    grows[e, sl] = grows[e, sl] * w
                return c2

            lax.fori_loop(0, CB, scale, 0)
            pltpu.sync_copy(grows, acc.at[rowv.at[ch]], add=True)
            return carry

        lax.fori_loop(0, nch, chunk, 0)

        plsc.subcore_barrier()

        for k in range(nstg):
            start = sid * rows_per_sub + k * rstg
            pltpu.sync_copy(acc.at[pl.ds(start, rstg)], zstg)
            pltpu.sync_copy(zstg, out_hbm.at[cid, pl.ds(start, rstg)])

    return spmm(feat, row3, col3, ev3)


def _normalize_tc(x):
    n_nodes, d = x.shape
    bm = 2000

    def body(x_ref, o_ref):
        xb = x_ref[...]
        s = jnp.sum(xb, axis=1, keepdims=True) + 0.0001
        o_ref[...] = xb / s

    return pl.pallas_call(
        body,
        grid=(n_nodes // bm,),
        in_specs=[pl.BlockSpec((bm, d), lambda i: (i, 0))],
        out_specs=pl.BlockSpec((bm, d), lambda i: (i, 0)),
        out_shape=jax.ShapeDtypeStruct((n_nodes, d), jnp.float32),
    )(x)


def _fused_linear_tc(partials, w, b, relu):
    _, n_nodes, d = partials.shape
    bm = 2000

    def body(p_ref, w_ref, b_ref, o_ref):
        a = p_ref[0] + p_ref[1]
        y = lax.dot_general(a, w_ref[...], (((1,), (1,)), ((), ())),
                            preferred_element_type=jnp.float32) + b_ref[...]
        o_ref[...] = jnp.maximum(y, 0.0) if relu else y

    return pl.pallas_call(
        body,
        grid=(n_nodes // bm,),
        in_specs=[pl.BlockSpec((NC, bm, d), lambda i: (0, i, 0)),
                  pl.BlockSpec((d, d), lambda i: (0, 0)),
                  pl.BlockSpec((1, d), lambda i: (0, 0))],
        out_specs=pl.BlockSpec((bm, d), lambda i: (i, 0)),
        out_shape=jax.ShapeDtypeStruct((n_nodes, d), jnp.float32),
    )(partials, w, b.reshape(1, d))


def kernel(x, edge_index, edge_values, W1, b1, W2, b2):
    n_nodes, d = x.shape
    n_edges = edge_index.shape[1]
    nw = NC * NS
    step = nw * CB
    e_pad = ((n_edges + step - 1) // step) * step
    row = edge_index[0]
    col = edge_index[1]
    ev = edge_values
    if e_pad != n_edges:
        pad = e_pad - n_edges
        row = jnp.concatenate([row, jnp.zeros((pad,), jnp.int32)])
        col = jnp.concatenate([col, jnp.zeros((pad,), jnp.int32)])
        ev = jnp.concatenate([ev, jnp.zeros((pad,), jnp.float32)])
    nch = e_pad // step
    row3 = row.reshape(nw, nch, CB)
    col3 = col.reshape(nw, nch, CB)
    ev3 = ev.reshape(nw, nch, CB)

    xn = _normalize_tc(x)
    p1 = _spmm_sc(xn, row3, col3, ev3)
    h = _fused_linear_tc(p1, W1, b1, relu=True)
    p2 = _spmm_sc(p2_in := h, row3, col3, ev3)
    y = _fused_linear_tc(p2, W2, b2, relu=False)
    return y


# speedup vs baseline: 4.1671x; 4.1671x over previous
import functools

import jax
import jax.numpy as jnp
from jax import lax
from jax.experimental import pallas as pl
from jax.experimental.pallas import tpu as pltpu
from jax.experimental.pallas import tpu_sc as plsc

NC = 2
NS = 16
LANES = 16
CB = 80


def _spmm_sc(feat, edges4, ev3):
    n_nodes, d = feat.shape
    nch = edges4.shape[1]
    rows_per_sub = (n_nodes // (NS * 8)) * 8
    rem = n_nodes - NS * rows_per_sub
    rstg = 64

    def _spans(length):
        out, off = [], 0
        while off < length:
            c = min(rstg, length - off)
            out.append((off, c))
            off += c
        return out

    mesh = plsc.VectorSubcoreMesh(core_axis_name="c", subcore_axis_name="s")

    @functools.partial(
        pl.kernel,
        out_type=jax.ShapeDtypeStruct((NC, n_nodes, d), jnp.float32),
        mesh=mesh,
        scratch_types=[
            pltpu.VMEM_SHARED((n_nodes, d), jnp.float32),
            pltpu.VMEM((2, CB), jnp.int32),
            pltpu.VMEM((CB,), jnp.float32),
            pltpu.VMEM((CB, d), jnp.float32),
            pltpu.VMEM((rstg, d), jnp.float32),
            pltpu.SemaphoreType.DMA,
        ],
    )
    def spmm(feat_hbm, edges_hbm, ev_hbm, out_hbm,
             acc, ebuf, evb, grows, zstg, sem):
        cid = lax.axis_index("c")
        sid = lax.axis_index("s")
        wid = sid * NC + cid

        zeros16 = jnp.zeros((LANES,), jnp.float32)

        def zb(i, carry):
            for j in range(d // LANES):
                zstg[i, pl.ds(j * LANES, LANES)] = zeros16
            return carry

        lax.fori_loop(0, rstg, zb, 0)
        for off, c in _spans(rows_per_sub):
            start = pl.multiple_of(sid * rows_per_sub + off, 8)
            pltpu.sync_copy(zstg.at[pl.ds(0, c)], acc.at[pl.ds(start, c)])
        if rem:
            @pl.when(sid == NS - 1)
            def _():
                for off, c in _spans(rem):
                    pltpu.sync_copy(
                        zstg.at[pl.ds(0, c)],
                        acc.at[pl.ds(NS * rows_per_sub + off, c)])

        plsc.subcore_barrier()

        def chunk(ch, carry):
            pltpu.sync_copy(edges_hbm.at[wid, ch], ebuf)
            pltpu.sync_copy(ev_hbm.at[wid, ch], evb)
            pltpu.async_copy(feat_hbm.at[ebuf.at[1]], grows, sem).wait()

            def scale(g, c2):
                wv = evb[pl.ds(g * LANES, LANES)]
                for l in range(LANES):
                    w = wv[l]
                    e = g * LANES + l
                    for j in range(d // LANES):
                        sl = pl.ds(j * LANES, LANES)
                        grows[e, sl] = grows[e, sl] * w
                return c2

            lax.fori_loop(0, CB // LANES, scale, 0)
            pltpu.sync_copy(grows, acc.at[ebuf.at[0]], add=True)
            return carry

        lax.fori_loop(0, nch, chunk, 0)

        plsc.subcore_barrier()

        for off, c in _spans(rows_per_sub):
            start = pl.multiple_of(sid * rows_per_sub + off, 8)
            pltpu.sync_copy(acc.at[pl.ds(start, c)], zstg.at[pl.ds(0, c)])
            pltpu.sync_copy(zstg.at[pl.ds(0, c)],
                            out_hbm.at[cid, pl.ds(start, c)])
        if rem:
            @pl.when(sid == NS - 1)
            def _():
                for off, c in _spans(rem):
                    start = NS * rows_per_sub + off
                    pltpu.sync_copy(acc.at[pl.ds(start, c)],
                                    zstg.at[pl.ds(0, c)])
                    pltpu.sync_copy(zstg.at[pl.ds(0, c)],
                                    out_hbm.at[cid, pl.ds(start, c)])

    return spmm(feat, edges4, ev3)


def _normalize_tc(x):
    n_nodes, d = x.shape
    bm = 2000

    def body(x_ref, o_ref):
        xb = x_ref[...]
        s = jnp.sum(xb, axis=1, keepdims=True) + 0.0001
        o_ref[...] = xb / s

    return pl.pallas_call(
        body,
        grid=(n_nodes // bm,),
        in_specs=[pl.BlockSpec((bm, d), lambda i: (i, 0))],
        out_specs=pl.BlockSpec((bm, d), lambda i: (i, 0)),
        out_shape=jax.ShapeDtypeStruct((n_nodes, d), jnp.float32),
    )(x)


def _fused_linear_tc(partials, w, b, relu):
    _, n_nodes, d = partials.shape
    bm = 2000

    def body(p_ref, w_ref, b_ref, o_ref):
        a = p_ref[0] + p_ref[1]
        y = lax.dot_general(a, w_ref[...], (((1,), (1,)), ((), ())),
                            preferred_element_type=jnp.float32) + b_ref[...]
        o_ref[...] = jnp.maximum(y, 0.0) if relu else y

    return pl.pallas_call(
        body,
        grid=(n_nodes // bm,),
        in_specs=[pl.BlockSpec((NC, bm, d), lambda i: (0, i, 0)),
                  pl.BlockSpec((d, d), lambda i: (0, 0)),
                  pl.BlockSpec((1, d), lambda i: (0, 0))],
        out_specs=pl.BlockSpec((bm, d), lambda i: (i, 0)),
        out_shape=jax.ShapeDtypeStruct((n_nodes, d), jnp.float32),
    )(partials, w, b.reshape(1, d))


def kernel(x, edge_index, edge_values, W1, b1, W2, b2):
    n_nodes, d = x.shape
    n_edges = edge_index.shape[1]
    nw = NC * NS
    step = nw * CB
    e_pad = ((n_edges + step - 1) // step) * step
    row = edge_index[0]
    col = edge_index[1]
    ev = edge_values
    if e_pad != n_edges:
        pad = e_pad - n_edges
        row = jnp.concatenate([row, jnp.zeros((pad,), jnp.int32)])
        col = jnp.concatenate([col, jnp.zeros((pad,), jnp.int32)])
        ev = jnp.concatenate([ev, jnp.zeros((pad,), jnp.float32)])
    nch = e_pad // step
    edges4 = jnp.stack(
        [row.reshape(nw, nch, CB), col.reshape(nw, nch, CB)], axis=2)
    ev3 = ev.reshape(nw, nch, CB)

    xn = _normalize_tc(x)
    p1 = _spmm_sc(xn, edges4, ev3)
    h = _fused_linear_tc(p1, W1, b1, relu=True)
    p2 = _spmm_sc(h, edges4, ev3)
    y = _fused_linear_tc(p2, W2, b2, relu=False)
    return y
